# baseline (speedup 1.0000x reference)
import functools

import jax
import jax.numpy as jnp
from jax.experimental import pallas as pl
from jax.experimental.pallas import tpu as pltpu

_F32 = jnp.float32
_BF16 = jnp.bfloat16


def _build_patches(xz, h, w, cin):
    slabs = []
    for e in range(3):
        for dj in range(3):
            s = xz[:, e:e + h, dj:dj + 2 * w]
            slabs.append(s.reshape(cin, h * 2 * w))
    return jnp.concatenate(slabs, axis=0)


def _stats_kernel(xz_ref, wt_ref, sum_ref, ssq_ref, *, h, w, cin):
    @pl.when(pl.program_id(1) == 0)
    def _():
        sum_ref[...] = jnp.zeros_like(sum_ref)
        ssq_ref[...] = jnp.zeros_like(ssq_ref)

    p = _build_patches(xz_ref[0], h, w, cin)
    y = jnp.dot(wt_ref[...], p, preferred_element_type=_F32)
    r_dim = y.shape[0]
    lr = sum_ref.shape[-1]
    y3 = y.reshape(r_dim, (h * 2 * w) // lr, lr)
    sum_ref[0] += jnp.sum(y3, axis=1)
    ssq_ref[0] += jnp.sum(y3 * y3, axis=1)


def _apply_kernel(xz_ref, wt_ref, scale_ref, shift_ref, o_ref, *, h, w, cin):
    p = _build_patches(xz_ref[0], h, w, cin)
    y = jnp.dot(wt_ref[...], p, preferred_element_type=_F32)
    y = y * scale_ref[0] + shift_ref[0]
    y = jnp.where(y >= 0, y, 0.2 * y)
    cout = y.shape[0] // 2
    a3 = y[:cout].reshape(cout, h, 2 * w)
    b3 = y[cout:].reshape(cout, h, 2 * w)
    v = jnp.stack([a3, b3], axis=2)
    o_ref[0] = v.reshape(cout, 2 * h, 2 * w).astype(o_ref.dtype)


@jax.jit
def _run(x_nchw, w_oihw, gamma, beta):
    n, cin, h, w = x_nchw.shape
    cout = w_oihw.shape[0]
    k_dim = 9 * cin
    r_dim = 2 * cout
    m_out = h * 2 * w

    xp = jnp.pad(x_nchw, ((0, 0), (0, 0), (1, 1), (1, 1)), mode="edge")
    one = 1.0 + 0.0 * gamma[0].astype(_F32)
    xz = (jnp.repeat(xp, 2, axis=3)[:, :, :, 1:2 * w + 3] * one).astype(_BF16)

    s_fold = jnp.array([[[1, 0, 0], [0, 1, 0], [0, 1, 0]],
                        [[0, 1, 0], [0, 1, 0], [0, 0, 1]]], _F32)
    w2 = jnp.einsum("pka,ockd->poadc", s_fold,
                    w_oihw.astype(_F32)).reshape(r_dim, k_dim)
    wt = w2.astype(_BF16)

    cores = 2
    per_core = n // cores
    lr = 128 if m_out % 128 == 0 else m_out

    xz_spec = pl.BlockSpec((1, cin, h + 2, 2 * w + 2),
                           lambda c, i: (c * per_core + i, 0, 0, 0))
    wt_spec = pl.BlockSpec((r_dim, k_dim), lambda c, i: (0, 0))
    stat_spec = pl.BlockSpec((1, r_dim, lr), lambda c, i: (c, 0, 0))

    col_sum, col_ssq = pl.pallas_call(
        functools.partial(_stats_kernel, h=h, w=w, cin=cin),
        out_shape=(jax.ShapeDtypeStruct((cores, r_dim, lr), _F32),
                   jax.ShapeDtypeStruct((cores, r_dim, lr), _F32)),
        grid=(cores, per_core),
        in_specs=[xz_spec, wt_spec],
        out_specs=(stat_spec, stat_spec),
        compiler_params=pltpu.CompilerParams(
            dimension_semantics=("parallel", "arbitrary")),
    )(xz, wt)
    col_sum = jnp.sum(col_sum, axis=(0, 2))
    col_ssq = jnp.sum(col_ssq, axis=(0, 2))

    count = jnp.asarray(2 * n * m_out, _F32)
    mean = jnp.sum(col_sum.reshape(2, cout), axis=0) / count
    var = jnp.maximum(
        jnp.sum(col_ssq.reshape(2, cout), axis=0) / count - mean * mean, 0.0)
    scale = gamma.astype(_F32) * jax.lax.rsqrt(var + 1e-5)
    shift = beta.astype(_F32) - mean * scale
    scale_c = jnp.tile(scale, 2).reshape(1, r_dim, 1)
    shift_c = jnp.tile(shift, 2).reshape(1, r_dim, 1)

    svec_spec = pl.BlockSpec((1, r_dim, 1), lambda c, i: (0, 0, 0))
    out = pl.pallas_call(
        functools.partial(_apply_kernel, h=h, w=w, cin=cin),
        out_shape=jax.ShapeDtypeStruct((n, cout, 2 * h, 2 * w), x_nchw.dtype),
        grid=(cores, per_core),
        in_specs=[xz_spec, wt_spec, svec_spec, svec_spec],
        out_specs=pl.BlockSpec((1, cout, 2 * h, 2 * w),
                               lambda c, i: (c * per_core + i, 0, 0, 0)),
        compiler_params=pltpu.CompilerParams(
            dimension_semantics=("parallel", "arbitrary")),
    )(xz, wt, scale_c, shift_c)
    return out


def kernel(x, conv_w, conv_b, bn_gamma, bn_beta):
    del conv_b
    return _run(x, conv_w, bn_gamma, bn_beta)


# speedup vs baseline: 3.0391x; 3.0391x over previous
import functools

import jax
import jax.numpy as jnp
from jax.experimental import pallas as pl
from jax.experimental.pallas import tpu as pltpu

_F32 = jnp.float32
_BF16 = jnp.bfloat16


def _build_patches(xz, h, w, cin):
    slabs = []
    for e in range(3):
        for dj in range(3):
            s = xz[:, e:e + h, dj:dj + 2 * w]
            slabs.append(s.reshape(cin, h * 2 * w))
    return jnp.concatenate(slabs, axis=0)


def _stats_kernel(x_ref, wc_ref, sum_ref, ssq_ref, *, h, w, cin):
    @pl.when(pl.program_id(1) == 0)
    def _():
        sum_ref[...] = jnp.zeros_like(sum_ref)
        ssq_ref[...] = jnp.zeros_like(ssq_ref)

    xv = x_ref[0].astype(_BF16)
    xr = jnp.concatenate([xv[:, :1], xv, xv[:, -1:]], axis=1)
    x3 = jnp.concatenate([xr[:, :, :1], xr, xr[:, :, -1:]], axis=2)
    slabs = []
    for a in range(3):
        for b in range(3):
            s_ = x3[:, a:a + h, b:b + w]
            slabs.append(s_.reshape(cin, h * w))
    p = jnp.concatenate(slabs, axis=0)
    y = jnp.dot(wc_ref[...], p, preferred_element_type=_F32)
    c_dim = y.shape[0]
    lr = sum_ref.shape[-1]
    y3 = y.reshape(c_dim, (h * w) // lr, lr)
    sum_ref[0] += jnp.sum(y3, axis=1)
    ssq_ref[0] += jnp.sum(y3 * y3, axis=1)


def _apply_kernel(xz_ref, wt_ref, scale_ref, shift_ref, o_ref, *, h, w, cin):
    p = _build_patches(xz_ref[0], h, w, cin)
    y = jnp.dot(wt_ref[...], p, preferred_element_type=_F32)
    y = y * scale_ref[0] + shift_ref[0]
    y = jnp.where(y >= 0, y, 0.2 * y)
    cout = y.shape[0] // 2
    a3 = y[:cout].reshape(cout, h, 2 * w)
    b3 = y[cout:].reshape(cout, h, 2 * w)
    v = jnp.stack([a3, b3], axis=2)
    o_ref[0] = v.reshape(cout, 2 * h, 2 * w).astype(o_ref.dtype)


@jax.jit
def _run(x_nchw, w_oihw, gamma, beta):
    n, cin, h, w = x_nchw.shape
    cout = w_oihw.shape[0]
    k_dim = 9 * cin
    r_dim = 2 * cout
    m_out = h * 2 * w

    xp = jnp.pad(x_nchw, ((0, 0), (0, 0), (1, 1), (1, 1)), mode="edge")
    xz = jnp.repeat(xp, 2, axis=3)[:, :, :, 1:2 * w + 3].astype(_BF16)

    s_fold = jnp.array([[[1, 0, 0], [0, 1, 0], [0, 1, 0]],
                        [[0, 1, 0], [0, 1, 0], [0, 0, 1]]], _F32)
    w2 = jnp.einsum("pka,ockd->poadc", s_fold,
                    w_oihw.astype(_F32)).reshape(r_dim, k_dim)
    wt = w2.astype(_BF16)
    wc4 = jnp.einsum("pha,qwb,ochw->pqoabc", s_fold, s_fold,
                     w_oihw.astype(_F32)).reshape(4 * cout, k_dim)
    wc4 = wc4.astype(_BF16)

    cores = 2
    per_core = n // cores
    lr = 128 if m_out % 128 == 0 else m_out
    lrs = 128 if (h * w) % 128 == 0 else h * w

    xz_spec = pl.BlockSpec((1, cin, h + 2, 2 * w + 2),
                           lambda c, i: (c * per_core + i, 0, 0, 0))
    x_spec = pl.BlockSpec((1, cin, h, w),
                          lambda c, i: (c * per_core + i, 0, 0, 0))
    wt_spec = pl.BlockSpec((r_dim, k_dim), lambda c, i: (0, 0))
    wc4_spec = pl.BlockSpec((4 * cout, k_dim), lambda c, i: (0, 0))
    stat_spec = pl.BlockSpec((1, 4 * cout, lrs), lambda c, i: (c, 0, 0))

    col_sum, col_ssq = pl.pallas_call(
        functools.partial(_stats_kernel, h=h, w=w, cin=cin),
        out_shape=(jax.ShapeDtypeStruct((cores, 4 * cout, lrs), _F32),
                   jax.ShapeDtypeStruct((cores, 4 * cout, lrs), _F32)),
        grid=(cores, per_core),
        in_specs=[x_spec, wc4_spec],
        out_specs=(stat_spec, stat_spec),
        compiler_params=pltpu.CompilerParams(
            dimension_semantics=("parallel", "arbitrary")),
    )(x_nchw, wc4)
    col_sum = jnp.sum(col_sum, axis=(0, 2))
    col_ssq = jnp.sum(col_ssq, axis=(0, 2))

    count = jnp.asarray(2 * n * m_out, _F32)
    mean = jnp.sum(col_sum.reshape(4, cout), axis=0) / count
    var = jnp.maximum(
        jnp.sum(col_ssq.reshape(4, cout), axis=0) / count - mean * mean, 0.0)
    scale = gamma.astype(_F32) * jax.lax.rsqrt(var + 1e-5)
    shift = beta.astype(_F32) - mean * scale
    scale_c = jnp.tile(scale, 2).reshape(1, r_dim, 1)
    shift_c = jnp.tile(shift, 2).reshape(1, r_dim, 1)

    svec_spec = pl.BlockSpec((1, r_dim, 1), lambda c, i: (0, 0, 0))
    out = pl.pallas_call(
        functools.partial(_apply_kernel, h=h, w=w, cin=cin),
        out_shape=jax.ShapeDtypeStruct((n, cout, 2 * h, 2 * w), x_nchw.dtype),
        grid=(cores, per_core),
        in_specs=[xz_spec, wt_spec, svec_spec, svec_spec],
        out_specs=pl.BlockSpec((1, cout, 2 * h, 2 * w),
                               lambda c, i: (c * per_core + i, 0, 0, 0)),
        compiler_params=pltpu.CompilerParams(
            dimension_semantics=("parallel", "arbitrary")),
    )(xz, wt, scale_c, shift_c)
    return out


def kernel(x, conv_w, conv_b, bn_gamma, bn_beta):
    del conv_b
    return _run(x, conv_w, bn_gamma, bn_beta)


# speedup vs baseline: 3.3713x; 1.1093x over previous
import functools

import jax
import jax.numpy as jnp
from jax.experimental import pallas as pl
from jax.experimental.pallas import tpu as pltpu

_F32 = jnp.float32
_BF16 = jnp.bfloat16


def _build_patches(xz, h, w, cin):
    slabs = []
    for e in range(3):
        for dj in range(3):
            s = xz[:, e:e + h, dj:dj + 2 * w]
            slabs.append(s.reshape(cin, h * 2 * w))
    return jnp.concatenate(slabs, axis=0)


def _stats_kernel(x_ref, wc_ref, sum_ref, ssq_ref, *, h, w, cin):
    @pl.when(pl.program_id(1) == 0)
    def _():
        sum_ref[...] = jnp.zeros_like(sum_ref)
        ssq_ref[...] = jnp.zeros_like(ssq_ref)

    xv = x_ref[0].astype(_BF16)
    xr = jnp.concatenate([xv[:, :1], xv, xv[:, -1:]], axis=1)
    x3 = jnp.concatenate([xr[:, :, :1], xr, xr[:, :, -1:]], axis=2)
    slabs = []
    for a in range(3):
        for b in range(3):
            s_ = x3[:, a:a + h, b:b + w]
            slabs.append(s_.reshape(cin, h * w))
    p = jnp.concatenate(slabs, axis=0)
    y = jnp.dot(wc_ref[...], p, preferred_element_type=_F32)
    c_dim = y.shape[0]
    lr = sum_ref.shape[-1]
    y3 = y.reshape(c_dim, (h * w) // lr, lr)
    sum_ref[0] += jnp.sum(y3, axis=1)
    ssq_ref[0] += jnp.sum(y3 * y3, axis=1)


def _apply_kernel(xz_ref, wt_ref, scale_ref, shift_ref, o_ref, *, h, w, cin):
    p = _build_patches(xz_ref[0], h, w, cin)
    y = jnp.dot(wt_ref[...], p, preferred_element_type=_F32)
    y = y * scale_ref[0] + shift_ref[0]
    y = jnp.where(y >= 0, y, 0.2 * y)
    cout = y.shape[0] // 2
    a3 = y[:cout].reshape(cout, h, 2 * w)
    b3 = y[cout:].reshape(cout, h, 2 * w)
    o_ref[0, :, :, 0, :] = a3.astype(o_ref.dtype)
    o_ref[0, :, :, 1, :] = b3.astype(o_ref.dtype)


@jax.jit
def _run(x_nchw, w_oihw, gamma, beta):
    n, cin, h, w = x_nchw.shape
    cout = w_oihw.shape[0]
    k_dim = 9 * cin
    r_dim = 2 * cout
    m_out = h * 2 * w

    xp = jnp.pad(x_nchw, ((0, 0), (0, 0), (1, 1), (1, 1)), mode="edge")
    xz = jnp.repeat(xp, 2, axis=3)[:, :, :, 1:2 * w + 3].astype(_BF16)

    s_fold = jnp.array([[[1, 0, 0], [0, 1, 0], [0, 1, 0]],
                        [[0, 1, 0], [0, 1, 0], [0, 0, 1]]], _F32)
    w2 = jnp.einsum("pka,ockd->poadc", s_fold,
                    w_oihw.astype(_F32)).reshape(r_dim, k_dim)
    wt = w2.astype(_BF16)
    wc4 = jnp.einsum("pha,qwb,ochw->pqoabc", s_fold, s_fold,
                     w_oihw.astype(_F32)).reshape(4 * cout, k_dim)
    wc4 = wc4.astype(_BF16)

    cores = 2
    per_core = n // cores
    lr = 128 if m_out % 128 == 0 else m_out
    lrs = 128 if (h * w) % 128 == 0 else h * w

    xz_spec = pl.BlockSpec((1, cin, h + 2, 2 * w + 2),
                           lambda c, i: (c * per_core + i, 0, 0, 0))
    x_spec = pl.BlockSpec((1, cin, h, w),
                          lambda c, i: (c * per_core + i, 0, 0, 0))
    wt_spec = pl.BlockSpec((r_dim, k_dim), lambda c, i: (0, 0))
    wc4_spec = pl.BlockSpec((4 * cout, k_dim), lambda c, i: (0, 0))
    stat_spec = pl.BlockSpec((1, 4 * cout, lrs), lambda c, i: (c, 0, 0))

    col_sum, col_ssq = pl.pallas_call(
        functools.partial(_stats_kernel, h=h, w=w, cin=cin),
        out_shape=(jax.ShapeDtypeStruct((cores, 4 * cout, lrs), _F32),
                   jax.ShapeDtypeStruct((cores, 4 * cout, lrs), _F32)),
        grid=(cores, per_core),
        in_specs=[x_spec, wc4_spec],
        out_specs=(stat_spec, stat_spec),
        compiler_params=pltpu.CompilerParams(
            dimension_semantics=("parallel", "arbitrary")),
    )(x_nchw, wc4)
    col_sum = jnp.sum(col_sum, axis=(0, 2))
    col_ssq = jnp.sum(col_ssq, axis=(0, 2))

    count = jnp.asarray(2 * n * m_out, _F32)
    mean = jnp.sum(col_sum.reshape(4, cout), axis=0) / count
    var = jnp.maximum(
        jnp.sum(col_ssq.reshape(4, cout), axis=0) / count - mean * mean, 0.0)
    scale = gamma.astype(_F32) * jax.lax.rsqrt(var + 1e-5)
    shift = beta.astype(_F32) - mean * scale
    scale_c = jnp.tile(scale, 2).reshape(1, r_dim, 1)
    shift_c = jnp.tile(shift, 2).reshape(1, r_dim, 1)

    svec_spec = pl.BlockSpec((1, r_dim, 1), lambda c, i: (0, 0, 0))
    out = pl.pallas_call(
        functools.partial(_apply_kernel, h=h, w=w, cin=cin),
        out_shape=jax.ShapeDtypeStruct((n, cout, h, 2, 2 * w), x_nchw.dtype),
        grid=(cores, per_core),
        in_specs=[xz_spec, wt_spec, svec_spec, svec_spec],
        out_specs=pl.BlockSpec((1, cout, h, 2, 2 * w),
                               lambda c, i: (c * per_core + i, 0, 0, 0, 0)),
        compiler_params=pltpu.CompilerParams(
            dimension_semantics=("parallel", "arbitrary")),
    )(xz, wt, scale_c, shift_c)
    return out.reshape(n, cout, 2 * h, 2 * w)


def kernel(x, conv_w, conv_b, bn_gamma, bn_beta):
    del conv_b
    return _run(x, conv_w, bn_gamma, bn_beta)


# speedup vs baseline: 3.3742x; 1.0009x over previous
import functools

import jax
import jax.numpy as jnp
from jax.experimental import pallas as pl
from jax.experimental.pallas import tpu as pltpu

_F32 = jnp.float32
_BF16 = jnp.bfloat16


def _build_patches(xz, h, w, cin):
    slabs = []
    for e in range(3):
        for dj in range(3):
            s = xz[:, e:e + h, dj:dj + 2 * w]
            slabs.append(s.reshape(cin, h * 2 * w))
    return jnp.concatenate(slabs, axis=0)


def _stats_kernel(x_ref, wc_ref, sum_ref, ssq_ref, *, h, w, cin):
    @pl.when(pl.program_id(1) == 0)
    def _():
        sum_ref[...] = jnp.zeros_like(sum_ref)
        ssq_ref[...] = jnp.zeros_like(ssq_ref)

    xv = x_ref[0].astype(_BF16)
    xr = jnp.concatenate([xv[:, :1], xv, xv[:, -1:]], axis=1)
    x3 = jnp.concatenate([xr[:, :, :1], xr, xr[:, :, -1:]], axis=2)
    slabs = []
    for a in range(3):
        for b in range(3):
            s_ = x3[:, a:a + h, b:b + w]
            slabs.append(s_.reshape(cin, h * w))
    p = jnp.concatenate(slabs, axis=0)
    y = jnp.dot(wc_ref[...], p, preferred_element_type=_F32)
    c_dim = y.shape[0]
    lr = sum_ref.shape[-1]
    y3 = y.reshape(c_dim, (h * w) // lr, lr)
    sum_ref[0] += jnp.sum(y3, axis=1)
    ssq_ref[0] += jnp.sum(y3 * y3, axis=1)


def _apply_kernel(xz_ref, wt_ref, scale_ref, shift_ref, o_ref, *, h, w, cin):
    p = _build_patches(xz_ref[0], h, w, cin)
    y = jnp.dot(wt_ref[...], p, preferred_element_type=_F32)
    y = y * scale_ref[0] + shift_ref[0]
    y = jnp.where(y >= 0, y, 0.2 * y)
    cout = y.shape[0] // 2
    a3 = y[:cout].reshape(cout, h, 2 * w)
    b3 = y[cout:].reshape(cout, h, 2 * w)
    o_ref[0, :, :, 0, :] = a3.astype(o_ref.dtype)
    o_ref[0, :, :, 1, :] = b3.astype(o_ref.dtype)


@jax.jit
def _run(x_nchw, w_oihw, gamma, beta):
    n, cin, h, w = x_nchw.shape
    cout = w_oihw.shape[0]
    k_dim = 9 * cin
    r_dim = 2 * cout
    m_out = h * 2 * w

    xp = jnp.pad(x_nchw.astype(_BF16), ((0, 0), (0, 0), (1, 1), (1, 1)),
                 mode="edge")
    xz = jnp.repeat(xp, 2, axis=3)[:, :, :, 1:2 * w + 3]

    s_fold = jnp.array([[[1, 0, 0], [0, 1, 0], [0, 1, 0]],
                        [[0, 1, 0], [0, 1, 0], [0, 0, 1]]], _F32)
    w2 = jnp.einsum("pka,ockd->poadc", s_fold,
                    w_oihw.astype(_F32)).reshape(r_dim, k_dim)
    wt = w2.astype(_BF16)
    wc4 = jnp.einsum("pha,qwb,ochw->pqoabc", s_fold, s_fold,
                     w_oihw.astype(_F32)).reshape(4 * cout, k_dim)
    wc4 = wc4.astype(_BF16)

    cores = 2
    per_core = n // cores
    lr = 128 if m_out % 128 == 0 else m_out
    lrs = 128 if (h * w) % 128 == 0 else h * w

    xz_spec = pl.BlockSpec((1, cin, h + 2, 2 * w + 2),
                           lambda c, i: (c * per_core + i, 0, 0, 0))
    x_spec = pl.BlockSpec((1, cin, h, w),
                          lambda c, i: (c * per_core + i, 0, 0, 0))
    wt_spec = pl.BlockSpec((r_dim, k_dim), lambda c, i: (0, 0))
    wc4_spec = pl.BlockSpec((4 * cout, k_dim), lambda c, i: (0, 0))
    stat_spec = pl.BlockSpec((1, 4 * cout, lrs), lambda c, i: (c, 0, 0))

    col_sum, col_ssq = pl.pallas_call(
        functools.partial(_stats_kernel, h=h, w=w, cin=cin),
        out_shape=(jax.ShapeDtypeStruct((cores, 4 * cout, lrs), _F32),
                   jax.ShapeDtypeStruct((cores, 4 * cout, lrs), _F32)),
        grid=(cores, per_core),
        in_specs=[x_spec, wc4_spec],
        out_specs=(stat_spec, stat_spec),
        compiler_params=pltpu.CompilerParams(
            dimension_semantics=("parallel", "arbitrary")),
    )(x_nchw, wc4)
    col_sum = jnp.sum(col_sum, axis=(0, 2))
    col_ssq = jnp.sum(col_ssq, axis=(0, 2))

    count = jnp.asarray(2 * n * m_out, _F32)
    mean = jnp.sum(col_sum.reshape(4, cout), axis=0) / count
    var = jnp.maximum(
        jnp.sum(col_ssq.reshape(4, cout), axis=0) / count - mean * mean, 0.0)
    scale = gamma.astype(_F32) * jax.lax.rsqrt(var + 1e-5)
    shift = beta.astype(_F32) - mean * scale
    scale_c = jnp.tile(scale, 2).reshape(1, r_dim, 1)
    shift_c = jnp.tile(shift, 2).reshape(1, r_dim, 1)

    svec_spec = pl.BlockSpec((1, r_dim, 1), lambda c, i: (0, 0, 0))
    out = pl.pallas_call(
        functools.partial(_apply_kernel, h=h, w=w, cin=cin),
        out_shape=jax.ShapeDtypeStruct((n, cout, h, 2, 2 * w), x_nchw.dtype),
        grid=(cores, per_core),
        in_specs=[xz_spec, wt_spec, svec_spec, svec_spec],
        out_specs=pl.BlockSpec((1, cout, h, 2, 2 * w),
                               lambda c, i: (c * per_core + i, 0, 0, 0, 0)),
        compiler_params=pltpu.CompilerParams(
            dimension_semantics=("parallel", "arbitrary")),
    )(xz, wt, scale_c, shift_c)
    return out.reshape(n, cout, 2 * h, 2 * w)


def kernel(x, conv_w, conv_b, bn_gamma, bn_beta):
    del conv_b
    return _run(x, conv_w, bn_gamma, bn_beta)
